# baseline (speedup 1.0000x reference)
import functools

import jax
import jax.numpy as jnp
from jax import lax
from jax.experimental import pallas as pl
from jax.experimental.pallas import tpu as pltpu
from jax.experimental.pallas import tpu_sc as plsc

_NC = 2
_NS = 16
_NW = _NC * _NS
_PAGE = 2 * 8 * 16 * 128
_HALF = _PAGE // 2


def _f16_bits(y):
    u = plsc.bitcast(y, jnp.int32)
    mag = u & 0x7FFFFFFF
    sign = lax.shift_right_logical(u, 16) & 0x8000
    lsb = lax.shift_right_logical(mag, 13) & 1
    hn = lax.shift_right_logical(mag + 0xFFF + lsb, 13) - 0x1C000
    z = plsc.bitcast(mag, jnp.float32) + 0.5
    hs = plsc.bitcast(z, jnp.int32) - 0x3F000000
    return sign | jnp.where(mag >= 0x38800000, hn, hs)


def _sc_body(cache, idxh, sclh, outh, idx_v, scl_v, page_v, out_v, sem):
    w = lax.axis_index("s") * _NC + lax.axis_index("c")
    base = w * 32
    pltpu.sync_copy(idxh.at[pl.ds(base, 32)], idx_v)
    pltpu.sync_copy(sclh.at[pl.ds(base, 32)], scl_v)

    iota = lax.iota(jnp.int32, 16)
    ev_base = iota * 2
    od_base = ev_base + 1

    def page_work(p, kvec, vvec, row):
        pltpu.async_copy(cache.at[idx_v.at[p]], page_v, sem).wait()

        def half(hbase, svec):
            def chunk(j, _):
                off = hbase + j * 32
                ev = plsc.load_gather(page_v.at[0], [off + ev_base])
                od = plsc.load_gather(page_v.at[0], [off + od_base])
                he = _f16_bits(ev * svec)
                ho = _f16_bits(od * svec)
                w32 = he | lax.shift_left(ho, 16)
                out_v[0, pl.ds(off, 32)] = plsc.bitcast(w32, jnp.int16)
                return 0

            lax.fori_loop(0, _HALF // 32, chunk, 0, unroll=2)

        half(0, kvec)
        half(_HALF, vvec)
        pltpu.sync_copy(out_v, outh.at[pl.ds(row, 1)])

    zero16 = jnp.zeros((16,), jnp.float32)

    @pl.when(w == 0)
    def _():
        page_work(0, zero16, zero16, 0)

    def loop(p, _):
        page_work(p, scl_v[p, 0, :], scl_v[p, 1, :], base + p + 1)
        return 0

    lax.fori_loop(0, 32, loop, 0)


def kernel(kv_cache, block_tables, k_scale, v_scale):
    num_blocks, _, H, bs, hd = kv_cache.shape
    B, M = block_tables.shape
    N = B * M
    flat = block_tables.reshape(-1).astype(jnp.int32)
    idx2d = flat.reshape(N, 1)
    valid = (flat > 0).astype(jnp.float32)
    base_scl = jnp.broadcast_to(
        jnp.stack([k_scale[0], v_scale[0]])[None, :, None], (N, 2, 16)
    )
    scl = base_scl * valid[:, None, None]
    cache2d = kv_cache.reshape(num_blocks, _PAGE)

    sc_call = pl.kernel(
        _sc_body,
        out_type=jax.ShapeDtypeStruct((N + 1, _PAGE), jnp.int16),
        mesh=plsc.VectorSubcoreMesh(core_axis_name="c", subcore_axis_name="s"),
        compiler_params=pltpu.CompilerParams(
            needs_layout_passes=False, use_tc_tiling_on_sc=False
        ),
        scratch_types=[
            pltpu.VMEM((32, 1), jnp.int32),
            pltpu.VMEM((32, 2, 16), jnp.float32),
            pltpu.VMEM((1, _PAGE), jnp.float32),
            pltpu.VMEM((1, _PAGE), jnp.int16),
            pltpu.SemaphoreType.DMA,
        ],
    )
    out = sc_call(cache2d, idx2d, scl)
    out = jax.lax.bitcast_convert_type(out, jnp.float16)
    return out.reshape(N + 1, 2, H, bs, hd)
